# baseline (device time: 158702 ns/iter reference)
import jax
import jax.numpy as jnp
from jax import lax
from jax.experimental import pallas as pl
from jax.experimental.pallas import tpu as pltpu


def kernel(x, A, B, C):
    Bb, S_loc, D = x.shape
    N = A.shape[1]

    A_t = A.T

    def body(x_ref, a_ref, b_ref, c_ref, y_ref,
             stage_ref, recv_ref, send_sem, recv_sem):
        my_x = lax.axis_index("x")
        my_y = lax.axis_index("y")

        barrier = pltpu.get_barrier_semaphore()
        pl.semaphore_signal(barrier, inc=1, device_id=(my_x, 1 - my_y),
                            device_id_type=pl.DeviceIdType.MESH)
        pl.semaphore_wait(barrier, 1)

        dA = jnp.exp(a_ref[:, :])[None]

        def step(t, h):
            xt = x_ref[:, t, :]
            bt = b_ref[:, t, :]
            ct = c_ref[:, t, :]
            h = h * dA + xt[:, None, :] * bt[:, :, None]
            y_ref[:, t, :] = jnp.sum(h * ct[:, :, None], axis=1)
            return h

        h = lax.fori_loop(0, S_loc, step, jnp.zeros((Bb, N, D), jnp.float32))
        stage_ref[...] = h

        rdma = pltpu.make_async_remote_copy(
            src_ref=stage_ref, dst_ref=recv_ref,
            send_sem=send_sem, recv_sem=recv_sem,
            device_id=(my_x, 1), device_id_type=pl.DeviceIdType.MESH)

        @pl.when(my_y == 0)
        def _():
            rdma.start()
            rdma.wait_send()

        @pl.when(my_y == 1)
        def _():
            rdma.wait_recv()

            def corr(t, g):
                g = g * dA
                ct = c_ref[:, t, :]
                y_ref[:, t, :] += jnp.sum(g * ct[:, :, None], axis=1)
                return g

            lax.fori_loop(0, S_loc, corr, recv_ref[...])

    return pl.pallas_call(
        body,
        out_shape=jax.ShapeDtypeStruct((Bb, S_loc, D), jnp.float32),
        in_specs=[pl.BlockSpec(memory_space=pltpu.VMEM)] * 4,
        out_specs=pl.BlockSpec(memory_space=pltpu.VMEM),
        scratch_shapes=[
            pltpu.VMEM((Bb, N, D), jnp.float32),
            pltpu.VMEM((Bb, N, D), jnp.float32),
            pltpu.SemaphoreType.DMA,
            pltpu.SemaphoreType.DMA,
        ],
        compiler_params=pltpu.CompilerParams(collective_id=0),
    )(x, A_t, B, C)


# device time: 95402 ns/iter; 1.6635x vs baseline; 1.6635x over previous
import os

import jax
import jax.numpy as jnp
from jax import lax
from jax.experimental import pallas as pl
from jax.experimental.pallas import tpu as pltpu

_L = 64
_STAGE = int(os.environ.get("V4_STAGE", "4"))


def kernel(x, A, B, C):
    Bb, S_loc, D = x.shape
    N = A.shape[1]
    L = _L
    Q = S_loc // 2
    nch = Q // L

    A_t = A.T

    def body(x_ref, a_ref, b_ref, c_ref, y_ref,
             hbuf_ref, yq_ref, yrecv_ref, stage_ref, strecv_ref,
             st_send_sems, st_recv_sems, y_send_sems, y_recv_sems):
        mx = lax.axis_index("x")
        my = lax.axis_index("y")
        myq = my * 2 + mx
        qlo = mx * Q
        nqlo = (1 - mx) * Q

        barrier = pltpu.get_barrier_semaphore()
        for tgt in [(1 - mx, my), (mx, 1 - my), (1 - mx, 1 - my)]:
            pl.semaphore_signal(barrier, inc=1, device_id=tgt,
                                device_id_type=pl.DeviceIdType.MESH)
        pl.semaphore_wait(barrier, 3)

        dA = jnp.exp(a_ref[:, :])
        pows = [dA]
        for _ in range(L - 1):
            pows.append(pows[-1] * dA)
        dApow = jnp.stack(pows)
        dA_L = pows[-1] * dA
        dA_2L = dA_L * dA_L
        dAq = dA_2L * dA_2L

        yrdmas = []
        h = jnp.zeros((Bb, N, D), jnp.float32)
        for ch in range(nch):
            slq = pl.ds(qlo + ch * L, L)
            sls = pl.ds(ch * L, L)
            xc = x_ref[:, slq, :]
            bc = b_ref[:, slq, :]
            cc = c_ref[:, slq, :]
            hbuf_ref[...] = xc[:, :, None, :] * bc[:, :, :, None]

            def stepl(l, hc):
                hc = hc * dA[None] + hbuf_ref[:, l]
                hbuf_ref[:, l] = hc
                return hc

            h = lax.fori_loop(0, L, stepl, h, unroll=8)
            yc = jnp.sum(hbuf_ref[...] * cc[:, :, :, None], axis=2)
            yq_ref[:, sls, :] = yc
            y_ref[:, slq, :] = yc
            if _STAGE >= 2:
                yr = pltpu.make_async_remote_copy(
                    src_ref=yq_ref.at[:, sls, :],
                    dst_ref=yrecv_ref.at[:, sls, :],
                    send_sem=y_send_sems.at[ch], recv_sem=y_recv_sems.at[ch],
                    device_id=(1 - mx, my),
                    device_id_type=pl.DeviceIdType.MESH)
                yr.start()
                yrdmas.append(yr)

        stage_ref[...] = h

        if _STAGE == 2:
            for yr in yrdmas:
                yr.wait_recv()
        if _STAGE <= 2:
            for yr in yrdmas:
                yr.wait_send()
            return

        def send_states(slot, targets):
            rs = []
            for i, tgt in enumerate(targets):
                r = pltpu.make_async_remote_copy(
                    src_ref=stage_ref, dst_ref=strecv_ref.at[slot],
                    send_sem=st_send_sems.at[i], recv_sem=st_recv_sems.at[slot],
                    device_id=tgt, device_id_type=pl.DeviceIdType.MESH)
                r.start()
                rs.append(r)
            for r in rs:
                r.wait_send()

        @pl.when(myq == 0)
        def _():
            send_states(0, [(1, 0), (0, 1), (1, 1)])

        @pl.when(myq == 1)
        def _():
            send_states(1, [(0, 1), (1, 1)])

        @pl.when(myq == 2)
        def _():
            send_states(2, [(1, 1)])

        for j in range(3):
            @pl.when(myq >= j + 1)
            def _(j=j):
                r = pltpu.make_async_remote_copy(
                    src_ref=stage_ref, dst_ref=strecv_ref.at[j],
                    send_sem=st_send_sems.at[j], recv_sem=st_recv_sems.at[j],
                    device_id=(mx, my), device_id_type=pl.DeviceIdType.MESH)
                r.wait_recv()

        s0 = jnp.where(myq == 0, stage_ref[...], strecv_ref[0])
        s1 = jnp.where(myq == 1, stage_ref[...], strecv_ref[1])
        s2 = jnp.where(myq == 2, stage_ref[...], strecv_ref[2])
        H0 = s0
        H1 = s1 + dAq[None] * H0
        H2 = s2 + dAq[None] * H1
        zero = jnp.zeros_like(H0)
        s_own = jnp.where(myq == 1, H0,
                          jnp.where(myq == 2, H1,
                                    jnp.where(myq == 3, H2, zero)))
        s_nbr = jnp.where(myq == 0, H0,
                          jnp.where(myq == 2, H2,
                                    jnp.where(myq == 3, H1, zero)))

        def correct(base, state, rdma_waits, add_recv):
            Pend = jnp.ones((N, D), jnp.float32)
            for ch in range(nch):
                if rdma_waits is not None:
                    rdma_waits[ch].wait_recv()
                sl = pl.ds(base + ch * L, L)
                cc = c_ref[:, sl, :]
                Pc = dApow * Pend[None]
                g = Pc[None] * state[:, None, :, :]
                corr = jnp.sum(g * cc[:, :, :, None], axis=2)
                if add_recv:
                    y_ref[:, sl, :] = (
                        yrecv_ref[:, pl.ds(ch * L, L), :] + corr)
                else:
                    y_ref[:, sl, :] += corr
                Pend = Pend * dA_L

        correct(qlo, s_own, None, add_recv=False)
        if _STAGE >= 4:
            correct(nqlo, s_nbr, yrdmas, add_recv=True)
        elif yrdmas:
            for yr in yrdmas:
                yr.wait_recv()

        for yr in yrdmas:
            yr.wait_send()

    return pl.pallas_call(
        body,
        out_shape=jax.ShapeDtypeStruct((Bb, S_loc, D), jnp.float32),
        in_specs=[pl.BlockSpec(memory_space=pltpu.VMEM)] * 4,
        out_specs=pl.BlockSpec(memory_space=pltpu.VMEM),
        scratch_shapes=[
            pltpu.VMEM((Bb, _L, N, D), jnp.float32),
            pltpu.VMEM((Bb, S_loc // 2, D), jnp.float32),
            pltpu.VMEM((Bb, S_loc // 2, D), jnp.float32),
            pltpu.VMEM((Bb, N, D), jnp.float32),
            pltpu.VMEM((3, Bb, N, D), jnp.float32),
            pltpu.SemaphoreType.DMA((3,)),
            pltpu.SemaphoreType.DMA((3,)),
            pltpu.SemaphoreType.DMA((4,)),
            pltpu.SemaphoreType.DMA((4,)),
        ],
        compiler_params=pltpu.CompilerParams(
            collective_id=0, vmem_limit_bytes=100 * 1024 * 1024),
    )(x, A_t, B, C)
